# baseline (speedup 1.0000x reference)
import functools

import jax
import jax.numpy as jnp
from jax import lax
from jax.experimental import pallas as pl
from jax.experimental.pallas import tpu as pltpu
from jax.experimental.pallas import tpu_sc as plsc

B, S, D, P, E = 4, 2048, 1024, 256, 8
N = B * S
T = 512
NT = N // T
G = NT + E - 1
XS_ROWS = G * T

NC, NS = 2, 16
NW = NC * NS
RPW = N // NW
XCH = 32
NXCH = RPW // XCH
OGRP = 4
OCH = XCH * OGRP
NOCH = RPW // OCH


@functools.cache
def _sc_kernels():
    mesh = plsc.VectorSubcoreMesh(core_axis_name="c", subcore_axis_name="s")

    @functools.partial(
        pl.kernel,
        mesh=mesh,
        compiler_params=pltpu.CompilerParams(needs_layout_passes=False),
        out_type=(
            jax.ShapeDtypeStruct((XS_ROWS, D), jnp.float32),
            jax.ShapeDtypeStruct((NW, NXCH, XCH), jnp.int32),
        ),
        scratch_types=[
            pltpu.VMEM((RPW,), jnp.int32),
            pltpu.VMEM((E, 16), jnp.int32),
            pltpu.VMEM((NXCH, XCH), jnp.int32),
            pltpu.VMEM((XCH, D), jnp.float32),
            pltpu.VMEM((XCH, D), jnp.float32),
            pltpu.SemaphoreType.DMA,
            pltpu.SemaphoreType.DMA,
            pltpu.SemaphoreType.DMA,
            pltpu.SemaphoreType.DMA,
            pltpu.SemaphoreType.DMA,
        ],
    )
    def scatter_x(x_hbm, eid_hbm, base_hbm, xs_hbm, pos_hbm,
                  ids_v, state_v, idx_v, rows0, rows1,
                  rs0, rs1, ws0, ws1, psem):
        wid = lax.axis_index("s") * NC + lax.axis_index("c")
        base = wid * RPW
        bufs = (rows0, rows1)
        rsem = (rs0, rs1)
        wsem = (ws0, ws1)

        def read(j, b):
            return pltpu.async_copy(
                x_hbm.at[pl.ds(base + j * XCH, XCH)], bufs[b], rsem[b])

        h_r = [read(0, 0), read(1, 1)]
        pltpu.sync_copy(eid_hbm.at[wid], ids_v)
        pltpu.sync_copy(base_hbm.at[wid], state_v)

        zero = jnp.zeros((16,), jnp.int32)
        state = [state_v[e, :] for e in range(E)]
        for k in range(RPW // 16):
            ids = ids_v[pl.ds(k * 16, 16)]
            pos16 = zero
            for e in range(E):
                m = ids == e
                c = plsc.cumsum(jnp.where(m, 1, 0))
                pos16 = pos16 + jnp.where(m, state[e] + c - 1, 0)
                state[e] = state[e] + plsc.all_reduce_population_count(m)
            idx_v[k // 2, pl.ds((k % 2) * 16, 16)] = pos16

        h_p = pltpu.async_copy(idx_v, pos_hbm.at[wid], psem)
        h_w = [None, None]
        for j in range(NXCH):
            b = j % 2
            h_r[b].wait()
            h_w[b] = pltpu.async_copy(bufs[b], xs_hbm.at[idx_v.at[j]], wsem[b])
            if j + 2 < NXCH:
                h_w[b].wait()
                h_r[b] = read(j + 2, b)
        h_w[(NXCH - 2) % 2].wait()
        h_w[(NXCH - 1) % 2].wait()
        h_p.wait()

    @functools.partial(
        pl.kernel,
        mesh=mesh,
        out_type=jax.ShapeDtypeStruct((N, P), jnp.float32),
        scratch_types=[
            pltpu.VMEM((NXCH, XCH), jnp.int32),
            pltpu.VMEM((OCH, P), jnp.float32),
            pltpu.VMEM((OCH, P), jnp.float32),
            pltpu.SemaphoreType.DMA,
            pltpu.SemaphoreType.DMA,
            pltpu.SemaphoreType.DMA,
            pltpu.SemaphoreType.DMA,
        ],
    )
    def gather_out(ys_hbm, pos_hbm, out_hbm, idx_v, rows0, rows1,
                   rs0, rs1, ws0, ws1):
        wid = lax.axis_index("s") * NC + lax.axis_index("c")
        base = wid * RPW
        pltpu.sync_copy(pos_hbm.at[wid], idx_v)
        bufs = (rows0, rows1)
        rsem = (rs0, rs1)
        wsem = (ws0, ws1)
        h_r = [[pltpu.async_copy(
                    ys_hbm.at[idx_v.at[j * OGRP + q]],
                    bufs[j].at[pl.ds(q * XCH, XCH)], rsem[j])
                for q in range(OGRP)]
               for j in range(NOCH)]
        h_w = []
        for j in range(NOCH):
            for h in h_r[j]:
                h.wait()
            h_w.append(pltpu.async_copy(
                bufs[j], out_hbm.at[pl.ds(base + j * OCH, OCH)], wsem[j]))
        for h in h_w:
            h.wait()

    return scatter_x, gather_out


def _mm_body(ei, xs_ref, w_ref, b_ref, out_ref):
    e = ei[pl.program_id(0)]
    xb = xs_ref[...].astype(jnp.bfloat16)
    y = lax.dot_general(
        xb, w_ref[e],
        dimension_numbers=(((1,), (0,)), ((), ())),
        preferred_element_type=jnp.float32,
    )
    out_ref[...] = y + b_ref[e]


_grouped_matmul = pl.pallas_call(
    _mm_body,
    grid_spec=pltpu.PrefetchScalarGridSpec(
        num_scalar_prefetch=1,
        grid=(G,),
        in_specs=[
            pl.BlockSpec((T, D), lambda g, ei: (g, 0)),
            pl.BlockSpec((E, D, P), lambda g, ei: (0, 0, 0)),
            pl.BlockSpec((E, 1, P), lambda g, ei: (0, 0, 0)),
        ],
        out_specs=pl.BlockSpec((T, P), lambda g, ei: (g, 0)),
    ),
    out_shape=jax.ShapeDtypeStruct((XS_ROWS, P), jnp.float32),
)


def _routing_metadata(idx_w):
    eids = jnp.arange(E, dtype=jnp.int32)
    onehot = (idx_w[:, :, None] == eids[None, None, :]).astype(jnp.int32)
    counts_wc = onehot.sum(1)
    counts = counts_wc.sum(0)
    tiles_e = (counts + T - 1) // T
    tile_start = jnp.cumsum(tiles_e) - tiles_e
    starts_pad = (tile_start * T).astype(jnp.int32)
    total_tiles = jnp.sum(tiles_e)

    wbase = jnp.cumsum(counts_wc, axis=0) - counts_wc
    base_w = jnp.broadcast_to(
        (starts_pad[None, :] + wbase)[:, :, None], (NW, E, 16)).astype(jnp.int32)

    g = jnp.arange(G, dtype=jnp.int32)
    in_e = ((g[:, None] >= tile_start[None, :])
            & (g[:, None] < (tile_start + tiles_e)[None, :]))
    e_of_g = jnp.sum(in_e.astype(jnp.int32) * eids[None, :], axis=1)
    expert_ids = jnp.where(g < total_tiles, e_of_g, E - 1).astype(jnp.int32)
    return base_w, expert_ids


def kernel(x, block_indices, W, b):
    xf = x.reshape(N, D)
    idx_w = block_indices.reshape(NW, RPW).astype(jnp.int32)
    base_w, expert_ids = _routing_metadata(idx_w)
    scatter_x, gather_out = _sc_kernels()
    xs, pos = scatter_x(xf, idx_w, base_w)
    ys = _grouped_matmul(expert_ids, xs,
                         W.astype(jnp.bfloat16).transpose(0, 2, 1),
                         b.reshape(E, 1, P))
    out = gather_out(ys, pos)
    return out.reshape(B, S, P)


# speedup vs baseline: 1.0482x; 1.0482x over previous
import functools

import jax
import jax.numpy as jnp
from jax import lax
from jax.experimental import pallas as pl
from jax.experimental.pallas import tpu as pltpu
from jax.experimental.pallas import tpu_sc as plsc

B, S, D, P, E = 4, 2048, 1024, 256, 8
N = B * S
T = 512
NT = N // T
G = NT + E - 1
XS_ROWS = G * T

NC, NS = 2, 16
NW = NC * NS
RPW = N // NW
XCH = 32
NXCH = RPW // XCH
OGRP = 4
OCH = XCH * OGRP
NOCH = RPW // OCH


@functools.cache
def _sc_kernels():
    mesh = plsc.VectorSubcoreMesh(core_axis_name="c", subcore_axis_name="s")

    @functools.partial(
        pl.kernel,
        mesh=mesh,
        compiler_params=pltpu.CompilerParams(needs_layout_passes=False),
        out_type=(
            jax.ShapeDtypeStruct((XS_ROWS, D), jnp.float32),
            jax.ShapeDtypeStruct((NW, NXCH, XCH), jnp.int32),
        ),
        scratch_types=[
            pltpu.VMEM((RPW,), jnp.int32),
            pltpu.VMEM((E, 16), jnp.int32),
            pltpu.VMEM((NXCH, XCH), jnp.int32),
            pltpu.VMEM((XCH, D), jnp.float32),
            pltpu.VMEM((XCH, D), jnp.float32),
            pltpu.SemaphoreType.DMA,
            pltpu.SemaphoreType.DMA,
            pltpu.SemaphoreType.DMA,
            pltpu.SemaphoreType.DMA,
            pltpu.SemaphoreType.DMA,
        ],
    )
    def scatter_x(x_hbm, eid_hbm, base_hbm, xs_hbm, pos_hbm,
                  ids_v, state_v, idx_v, rows0, rows1,
                  rs0, rs1, ws0, ws1, psem):
        wid = lax.axis_index("s") * NC + lax.axis_index("c")
        base = wid * RPW
        bufs = (rows0, rows1)
        rsem = (rs0, rs1)
        wsem = (ws0, ws1)

        def read(j, b):
            return pltpu.async_copy(
                x_hbm.at[pl.ds(base + j * XCH, XCH)], bufs[b], rsem[b])

        h_r = [read(0, 0), read(1, 1)]
        pltpu.sync_copy(eid_hbm.at[wid], ids_v)
        pltpu.sync_copy(base_hbm.at[wid], state_v)

        zero = jnp.zeros((16,), jnp.int32)
        state = [state_v[e, :] for e in range(E)]
        for k in range(RPW // 16):
            ids = ids_v[pl.ds(k * 16, 16)]
            pos16 = zero
            for e in range(E):
                m = ids == e
                c = plsc.cumsum(jnp.where(m, 1, 0))
                pos16 = pos16 + jnp.where(m, state[e] + c - 1, 0)
                state[e] = state[e] + plsc.all_reduce_population_count(m)
            idx_v[k // 2, pl.ds((k % 2) * 16, 16)] = pos16

        h_p = pltpu.async_copy(idx_v, pos_hbm.at[wid], psem)
        h_w = [None, None]
        for j in range(NXCH):
            b = j % 2
            h_r[b].wait()
            h_w[b] = pltpu.async_copy(bufs[b], xs_hbm.at[idx_v.at[j]], wsem[b])
            if j + 2 < NXCH:
                h_w[b].wait()
                h_r[b] = read(j + 2, b)
        h_w[(NXCH - 2) % 2].wait()
        h_w[(NXCH - 1) % 2].wait()
        h_p.wait()

    @functools.partial(
        pl.kernel,
        mesh=mesh,
        out_type=jax.ShapeDtypeStruct((N, P), jnp.float32),
        scratch_types=[
            pltpu.VMEM((NXCH, XCH), jnp.int32),
            pltpu.VMEM((OCH, P), jnp.float32),
            pltpu.VMEM((OCH, P), jnp.float32),
            pltpu.SemaphoreType.DMA,
            pltpu.SemaphoreType.DMA,
            pltpu.SemaphoreType.DMA,
            pltpu.SemaphoreType.DMA,
        ],
    )
    def gather_out(ys_hbm, pos_hbm, out_hbm, idx_v, rows0, rows1,
                   rs0, rs1, ws0, ws1):
        wid = lax.axis_index("s") * NC + lax.axis_index("c")
        base = wid * RPW
        pltpu.sync_copy(pos_hbm.at[wid], idx_v)
        bufs = (rows0, rows1)
        rsem = (rs0, rs1)
        wsem = (ws0, ws1)
        h_r = [[pltpu.async_copy(
                    ys_hbm.at[idx_v.at[j * OGRP + q]],
                    bufs[j].at[pl.ds(q * XCH, XCH)], rsem[j])
                for q in range(OGRP)]
               for j in range(NOCH)]
        h_w = []
        for j in range(NOCH):
            for h in h_r[j]:
                h.wait()
            h_w.append(pltpu.async_copy(
                bufs[j], out_hbm.at[pl.ds(base + j * OCH, OCH)], wsem[j]))
        for h in h_w:
            h.wait()

    return scatter_x, gather_out


def _mm_body(ei, ts, ir, xs_ref, w_ref, b_ref, out_ref):
    g = pl.program_id(0)

    @pl.when(ir[g] == 1)
    def _():
        e = ei[g]
        xb = xs_ref[...].astype(jnp.bfloat16)
        y = lax.dot_general(
            xb, w_ref[e],
            dimension_numbers=(((1,), (0,)), ((), ())),
            preferred_element_type=jnp.float32,
        )
        out_ref[...] = y + b_ref[e]


_grouped_matmul = pl.pallas_call(
    _mm_body,
    grid_spec=pltpu.PrefetchScalarGridSpec(
        num_scalar_prefetch=3,
        grid=(G,),
        in_specs=[
            pl.BlockSpec((T, D), lambda g, ei, ts, ir: (ts[g], 0)),
            pl.BlockSpec((E, D, P), lambda g, ei, ts, ir: (0, 0, 0)),
            pl.BlockSpec((E, 1, P), lambda g, ei, ts, ir: (0, 0, 0)),
        ],
        out_specs=pl.BlockSpec((T, P), lambda g, ei, ts, ir: (ts[g], 0)),
    ),
    out_shape=jax.ShapeDtypeStruct((XS_ROWS, P), jnp.float32),
)


def _routing_metadata(idx_w):
    eids = jnp.arange(E, dtype=jnp.int32)
    onehot = (idx_w[:, :, None] == eids[None, None, :]).astype(jnp.int32)
    counts_wc = onehot.sum(1)
    counts = counts_wc.sum(0)
    tiles_e = (counts + T - 1) // T
    tile_start = jnp.cumsum(tiles_e) - tiles_e
    starts_pad = (tile_start * T).astype(jnp.int32)
    total_tiles = jnp.sum(tiles_e)

    wbase = jnp.cumsum(counts_wc, axis=0) - counts_wc
    base_w = jnp.broadcast_to(
        (starts_pad[None, :] + wbase)[:, :, None], (NW, E, 16)).astype(jnp.int32)

    g = jnp.arange(G, dtype=jnp.int32)
    in_e = ((g[:, None] >= tile_start[None, :])
            & (g[:, None] < (tile_start + tiles_e)[None, :]))
    e_of_g = jnp.sum(in_e.astype(jnp.int32) * eids[None, :], axis=1)
    is_real = (g < total_tiles).astype(jnp.int32)
    expert_ids = jnp.where(is_real == 1, e_of_g, E - 1).astype(jnp.int32)
    tile_src = jnp.where(is_real == 1, g, total_tiles - 1).astype(jnp.int32)
    return base_w, expert_ids, tile_src, is_real


def kernel(x, block_indices, W, b):
    xf = x.reshape(N, D)
    idx_w = block_indices.reshape(NW, RPW).astype(jnp.int32)
    base_w, expert_ids, tile_src, is_real = _routing_metadata(idx_w)
    scatter_x, gather_out = _sc_kernels()
    xs, pos = scatter_x(xf, idx_w, base_w)
    ys = _grouped_matmul(expert_ids, tile_src, is_real, xs,
                         W.astype(jnp.bfloat16).transpose(0, 2, 1),
                         b.reshape(E, 1, P))
    out = gather_out(ys, pos)
    return out.reshape(B, S, P)


# speedup vs baseline: 1.0542x; 1.0057x over previous
import functools

import jax
import jax.numpy as jnp
from jax import lax
from jax.experimental import pallas as pl
from jax.experimental.pallas import tpu as pltpu
from jax.experimental.pallas import tpu_sc as plsc

B, S, D, P, E = 4, 2048, 1024, 256, 8
N = B * S
T = 512
NT = N // T
G = NT + E - 1
XS_ROWS = G * T

NC, NS = 2, 16
NW = NC * NS
RPW = N // NW
XCH = 32
NXCH = RPW // XCH
OGRP = 4
OCH = XCH * OGRP
NOCH = RPW // OCH


@functools.cache
def _sc_kernels():
    mesh = plsc.VectorSubcoreMesh(core_axis_name="c", subcore_axis_name="s")

    @functools.partial(
        pl.kernel,
        mesh=mesh,
        compiler_params=pltpu.CompilerParams(needs_layout_passes=False),
        out_type=(
            jax.ShapeDtypeStruct((XS_ROWS, D), jnp.float32),
            jax.ShapeDtypeStruct((NW, NXCH, XCH), jnp.int32),
        ),
        scratch_types=[
            pltpu.VMEM((RPW,), jnp.int32),
            pltpu.VMEM((E, 16), jnp.int32),
            pltpu.VMEM((NXCH, XCH), jnp.int32),
            pltpu.VMEM((XCH, D), jnp.float32),
            pltpu.VMEM((XCH, D), jnp.float32),
            pltpu.VMEM((XCH, D), jnp.float32),
            pltpu.SemaphoreType.DMA,
            pltpu.SemaphoreType.DMA,
            pltpu.SemaphoreType.DMA,
            pltpu.SemaphoreType.DMA,
            pltpu.SemaphoreType.DMA,
            pltpu.SemaphoreType.DMA,
            pltpu.SemaphoreType.DMA,
        ],
    )
    def scatter_x(x_hbm, eid_hbm, base_hbm, xs_hbm, pos_hbm,
                  ids_v, state_v, idx_v, rows0, rows1, rows2,
                  rs0, rs1, rs2, ws0, ws1, ws2, psem):
        wid = lax.axis_index("s") * NC + lax.axis_index("c")
        base = wid * RPW
        bufs = (rows0, rows1, rows2)
        rsem = (rs0, rs1, rs2)
        wsem = (ws0, ws1, ws2)
        NB = 3

        def read(j, b):
            return pltpu.async_copy(
                x_hbm.at[pl.ds(base + j * XCH, XCH)], bufs[b], rsem[b])

        h_r = [read(b, b) for b in range(NB)]
        pltpu.sync_copy(eid_hbm.at[wid], ids_v)
        pltpu.sync_copy(base_hbm.at[wid], state_v)

        zero = jnp.zeros((16,), jnp.int32)
        state = [state_v[e, :] for e in range(E)]
        for k in range(RPW // 16):
            ids = ids_v[pl.ds(k * 16, 16)]
            pos16 = zero
            for e in range(E):
                m = ids == e
                c = plsc.cumsum(jnp.where(m, 1, 0))
                pos16 = pos16 + jnp.where(m, state[e] + c - 1, 0)
                state[e] = state[e] + plsc.all_reduce_population_count(m)
            idx_v[k // 2, pl.ds((k % 2) * 16, 16)] = pos16

        h_p = pltpu.async_copy(idx_v, pos_hbm.at[wid], psem)
        h_w = [None] * NB
        for j in range(NXCH):
            b = j % NB
            h_r[b].wait()
            h_w[b] = pltpu.async_copy(bufs[b], xs_hbm.at[idx_v.at[j]], wsem[b])
            if j + NB < NXCH:
                h_w[b].wait()
                h_r[b] = read(j + NB, b)
        for b in range(NB):
            if h_w[b] is not None:
                h_w[b].wait()
        h_p.wait()

    @functools.partial(
        pl.kernel,
        mesh=mesh,
        out_type=jax.ShapeDtypeStruct((N, P), jnp.float32),
        scratch_types=[
            pltpu.VMEM((NXCH, XCH), jnp.int32),
            pltpu.VMEM((OCH, P), jnp.float32),
            pltpu.VMEM((OCH, P), jnp.float32),
            pltpu.SemaphoreType.DMA,
            pltpu.SemaphoreType.DMA,
            pltpu.SemaphoreType.DMA,
            pltpu.SemaphoreType.DMA,
        ],
    )
    def gather_out(ys_hbm, pos_hbm, out_hbm, idx_v, rows0, rows1,
                   rs0, rs1, ws0, ws1):
        wid = lax.axis_index("s") * NC + lax.axis_index("c")
        base = wid * RPW
        pltpu.sync_copy(pos_hbm.at[wid], idx_v)
        bufs = (rows0, rows1)
        rsem = (rs0, rs1)
        wsem = (ws0, ws1)
        h_r = [[pltpu.async_copy(
                    ys_hbm.at[idx_v.at[j * OGRP + q]],
                    bufs[j].at[pl.ds(q * XCH, XCH)], rsem[j])
                for q in range(OGRP)]
               for j in range(NOCH)]
        h_w = []
        for j in range(NOCH):
            for h in h_r[j]:
                h.wait()
            h_w.append(pltpu.async_copy(
                bufs[j], out_hbm.at[pl.ds(base + j * OCH, OCH)], wsem[j]))
        for h in h_w:
            h.wait()

    return scatter_x, gather_out


def _mm_body(ei, ts, ir, xs_ref, w_ref, b_ref, out_ref):
    g = pl.program_id(0)

    @pl.when(ir[g] == 1)
    def _():
        e = ei[g]
        xb = xs_ref[...].astype(jnp.bfloat16)
        y = lax.dot_general(
            xb, w_ref[e],
            dimension_numbers=(((1,), (0,)), ((), ())),
            preferred_element_type=jnp.float32,
        )
        out_ref[...] = y + b_ref[e]


_grouped_matmul = pl.pallas_call(
    _mm_body,
    grid_spec=pltpu.PrefetchScalarGridSpec(
        num_scalar_prefetch=3,
        grid=(G,),
        in_specs=[
            pl.BlockSpec((T, D), lambda g, ei, ts, ir: (ts[g], 0)),
            pl.BlockSpec((E, D, P), lambda g, ei, ts, ir: (0, 0, 0)),
            pl.BlockSpec((E, 1, P), lambda g, ei, ts, ir: (0, 0, 0)),
        ],
        out_specs=pl.BlockSpec((T, P), lambda g, ei, ts, ir: (ts[g], 0)),
    ),
    out_shape=jax.ShapeDtypeStruct((XS_ROWS, P), jnp.float32),
)


def _routing_metadata(idx_w):
    eids = jnp.arange(E, dtype=jnp.int32)
    onehot = (idx_w[:, :, None] == eids[None, None, :]).astype(jnp.int32)
    counts_wc = onehot.sum(1)
    counts = counts_wc.sum(0)
    tiles_e = (counts + T - 1) // T
    tile_start = jnp.cumsum(tiles_e) - tiles_e
    starts_pad = (tile_start * T).astype(jnp.int32)
    total_tiles = jnp.sum(tiles_e)

    wbase = jnp.cumsum(counts_wc, axis=0) - counts_wc
    base_w = jnp.broadcast_to(
        (starts_pad[None, :] + wbase)[:, :, None], (NW, E, 16)).astype(jnp.int32)

    g = jnp.arange(G, dtype=jnp.int32)
    in_e = ((g[:, None] >= tile_start[None, :])
            & (g[:, None] < (tile_start + tiles_e)[None, :]))
    e_of_g = jnp.sum(in_e.astype(jnp.int32) * eids[None, :], axis=1)
    is_real = (g < total_tiles).astype(jnp.int32)
    expert_ids = jnp.where(is_real == 1, e_of_g, E - 1).astype(jnp.int32)
    tile_src = jnp.where(is_real == 1, g, total_tiles - 1).astype(jnp.int32)
    return base_w, expert_ids, tile_src, is_real


def kernel(x, block_indices, W, b):
    xf = x.reshape(N, D)
    idx_w = block_indices.reshape(NW, RPW).astype(jnp.int32)
    base_w, expert_ids, tile_src, is_real = _routing_metadata(idx_w)
    scatter_x, gather_out = _sc_kernels()
    xs, pos = scatter_x(xf, idx_w, base_w)
    ys = _grouped_matmul(expert_ids, tile_src, is_real, xs,
                         W.astype(jnp.bfloat16).transpose(0, 2, 1),
                         b.reshape(E, 1, P))
    out = gather_out(ys, pos)
    return out.reshape(B, S, P)
